# Initial kernel scaffold; baseline (speedup 1.0000x reference)
import jax
import jax.numpy as jnp
from jax.experimental import pallas as pl


def kernel(users, items, enti_emd, UEnet, IEnet):
    raise NotImplementedError("write your pallas kernel here")


# speedup vs baseline: 1.2398x; 1.2398x over previous
import functools

import jax
import jax.numpy as jnp
from jax import lax
from jax.experimental import pallas as pl
from jax.experimental.pallas import tpu as pltpu
from jax.experimental.pallas import tpu_sc as plsc

_NC = 2
_NS = 16
_CH = 16


def _sc_gather(users, items, UEnet, IEnet):
    B = users.shape[0]
    E = UEnet.shape[1]
    NW = _NC * _NS
    b_per_w = B // NW
    n_ch = b_per_w // _CH
    mesh = plsc.VectorSubcoreMesh(core_axis_name="c", subcore_axis_name="s")

    @functools.partial(
        pl.kernel,
        out_type=(
            jax.ShapeDtypeStruct((B, E), jnp.float32),
            jax.ShapeDtypeStruct((B, E), jnp.float32),
        ),
        mesh=mesh,
        scratch_types=[
            pltpu.VMEM((b_per_w,), jnp.int32),
            pltpu.VMEM((b_per_w,), jnp.int32),
            pltpu.VMEM((_CH, E), jnp.float32),
            pltpu.SemaphoreType.DMA,
        ],
    )
    def gather_kernel(users_hbm, items_hbm, ue_hbm, ie_hbm, ug_out, ig_out,
                      uidx_v, iidx_v, rows_v, sem):
        wid = lax.axis_index("s") * _NC + lax.axis_index("c")
        base = wid * b_per_w
        pltpu.sync_copy(users_hbm.at[pl.ds(base, b_per_w)], uidx_v)
        pltpu.sync_copy(items_hbm.at[pl.ds(base, b_per_w)], iidx_v)

        def body(c, carry):
            off = c * _CH
            pltpu.async_copy(ue_hbm.at[uidx_v.at[pl.ds(off, _CH)]], rows_v,
                             sem).wait()
            pltpu.sync_copy(rows_v, ug_out.at[pl.ds(base + off, _CH)])
            pltpu.async_copy(ie_hbm.at[iidx_v.at[pl.ds(off, _CH)]], rows_v,
                             sem).wait()
            pltpu.sync_copy(rows_v, ig_out.at[pl.ds(base + off, _CH)])
            return carry

        lax.fori_loop(0, n_ch, body, 0)

    return gather_kernel(users, items, UEnet, IEnet)


def _tc_score(Ug, Ig, emd):
    B, E = Ug.shape
    D = emd.shape[1]
    BB = 256

    def body(ug_ref, ig_ref, e_ref, o_ref):
        pu = jnp.dot(ug_ref[...], e_ref[...], preferred_element_type=jnp.float32)
        pi = jnp.dot(ig_ref[...], e_ref[...], preferred_element_type=jnp.float32)
        s = jnp.sum(pu * pi, axis=1)
        o_ref[...] = jax.nn.sigmoid(s)

    return pl.pallas_call(
        body,
        grid=(B // BB,),
        in_specs=[
            pl.BlockSpec((BB, E), lambda i: (i, 0)),
            pl.BlockSpec((BB, E), lambda i: (i, 0)),
            pl.BlockSpec((E, D), lambda i: (0, 0)),
        ],
        out_specs=pl.BlockSpec((BB,), lambda i: (i,)),
        out_shape=jax.ShapeDtypeStruct((B,), jnp.float32),
    )(Ug, Ig, emd)


def kernel(users, items, enti_emd, UEnet, IEnet):
    Ug, Ig = _sc_gather(users, items, UEnet, IEnet)
    return _tc_score(Ug, Ig, enti_emd)


# speedup vs baseline: 1.2536x; 1.0111x over previous
import functools

import jax
import jax.numpy as jnp
from jax import lax
from jax.experimental import pallas as pl
from jax.experimental.pallas import tpu as pltpu
from jax.experimental.pallas import tpu_sc as plsc

_NC = 2
_NS = 16
_CH = 8


def _sc_gather(users, items, UEnet, IEnet):
    B = users.shape[0]
    E = UEnet.shape[1]
    NW = _NC * _NS
    b_per_w = B // NW
    n_ch = b_per_w // _CH
    mesh = plsc.VectorSubcoreMesh(core_axis_name="c", subcore_axis_name="s")

    @functools.partial(
        pl.kernel,
        out_type=(
            jax.ShapeDtypeStruct((B, E), jnp.float32),
            jax.ShapeDtypeStruct((B, E), jnp.float32),
        ),
        mesh=mesh,
        scratch_types=[
            pltpu.VMEM((b_per_w,), jnp.int32),
            pltpu.VMEM((b_per_w,), jnp.int32),
            pltpu.VMEM((_CH, E), jnp.float32),
            pltpu.VMEM((_CH, E), jnp.float32),
            pltpu.SemaphoreType.DMA,
            pltpu.SemaphoreType.DMA,
            pltpu.SemaphoreType.DMA,
            pltpu.SemaphoreType.DMA,
        ],
    )
    def gather_kernel(users_hbm, items_hbm, ue_hbm, ie_hbm, ug_out, ig_out,
                      uidx_v, iidx_v, buf0, buf1, g0, g1, s0, s1):
        wid = lax.axis_index("s") * _NC + lax.axis_index("c")
        base = wid * b_per_w
        pltpu.sync_copy(users_hbm.at[pl.ds(base, b_per_w)], uidx_v)
        pltpu.sync_copy(items_hbm.at[pl.ds(base, b_per_w)], iidx_v)

        bufs = (buf0, buf1)
        gsems = (g0, g1)
        ssems = (s0, s1)
        chunks = []
        for c in range(n_ch):
            chunks.append((ue_hbm, uidx_v, ug_out, c * _CH))
            chunks.append((ie_hbm, iidx_v, ig_out, c * _CH))
        n = len(chunks)

        def start_gather(c):
            tab, idxr, _, off = chunks[c]
            b = c & 1
            return pltpu.async_copy(tab.at[idxr.at[pl.ds(off, _CH)]],
                                    bufs[b], gsems[b])

        gat = [None, None]
        scat = [None, None]
        gat[0] = start_gather(0)
        for c in range(n):
            b = c & 1
            nb = (c + 1) & 1
            gat[b].wait()
            if c + 1 < n:
                if scat[nb] is not None:
                    scat[nb].wait()
                gat[nb] = start_gather(c + 1)
            _, _, outr, off = chunks[c]
            scat[b] = pltpu.make_async_copy(
                bufs[b], outr.at[pl.ds(base + off, _CH)], ssems[b])
            scat[b].start()
        scat[0].wait()
        scat[1].wait()

    return gather_kernel(users, items, UEnet, IEnet)


def _tc_score(Ug, Ig, emd):
    B, E = Ug.shape
    D = emd.shape[1]
    BB = 256

    def body(ug_ref, ig_ref, e_ref, o_ref):
        pu = jnp.dot(ug_ref[...], e_ref[...], preferred_element_type=jnp.float32)
        pi = jnp.dot(ig_ref[...], e_ref[...], preferred_element_type=jnp.float32)
        s = jnp.sum(pu * pi, axis=1)
        o_ref[...] = jax.nn.sigmoid(s)

    return pl.pallas_call(
        body,
        grid=(B // BB,),
        in_specs=[
            pl.BlockSpec((BB, E), lambda i: (i, 0)),
            pl.BlockSpec((BB, E), lambda i: (i, 0)),
            pl.BlockSpec((E, D), lambda i: (0, 0)),
        ],
        out_specs=pl.BlockSpec((BB,), lambda i: (i,)),
        out_shape=jax.ShapeDtypeStruct((B,), jnp.float32),
    )(Ug, Ig, emd)


def kernel(users, items, enti_emd, UEnet, IEnet):
    Ug, Ig = _sc_gather(users, items, UEnet, IEnet)
    return _tc_score(Ug, Ig, enti_emd)
